# baseline (speedup 1.0000x reference)
import jax
import jax.numpy as jnp
from jax import lax
from jax.experimental import pallas as pl
from jax.experimental.pallas import tpu as pltpu

N = 4096
NFEAT = 256
NHID = 128
TAU = 0.5
BLK = 512
NB = N // BLK


def _mega_body(adj1_hbm, adj2_hbm, clm_hbm,
               x1_ref, x2_ref, w11_ref, w21_ref, w12_ref, w22_ref,
               b11_ref, b21_ref, b12_ref, b22_ref,
               z1_ref, z2_ref, acc_ref,
               bufs_ref, adjv_ref, s21_ref, s22_ref, zn2_ref, sems_ref):
    i = pl.program_id(0)
    slot = lax.rem(i, 2)

    H = BLK // 2

    def issue(j, sl):
        @pl.when(j < NB)
        def _():
            pltpu.make_async_copy(
                adj2_hbm.at[pl.ds(j * BLK, H), :],
                bufs_ref.at[sl, pl.ds(0, H)], sems_ref.at[sl, 0]).start()
            pltpu.make_async_copy(
                adj2_hbm.at[pl.ds(j * BLK + H, H), :],
                bufs_ref.at[sl, pl.ds(H, H)], sems_ref.at[sl, 1]).start()

        @pl.when((NB <= j) & (j < 2 * NB))
        def _():
            pltpu.make_async_copy(
                adj1_hbm.at[pl.ds((j - NB) * BLK, H), :],
                bufs_ref.at[sl, pl.ds(0, H)], sems_ref.at[sl, 0]).start()
            pltpu.make_async_copy(
                adj1_hbm.at[pl.ds((j - NB) * BLK + H, H), :],
                bufs_ref.at[sl, pl.ds(H, H)], sems_ref.at[sl, 1]).start()

        @pl.when(2 * NB <= j)
        def _():
            pltpu.make_async_copy(
                clm_hbm.at[pl.ds((j - 2 * NB) * BLK, H), :],
                bufs_ref.at[sl, pl.ds(0, H)], sems_ref.at[sl, 0]).start()
            pltpu.make_async_copy(
                clm_hbm.at[pl.ds((j - 2 * NB) * BLK + H, H), :],
                bufs_ref.at[sl, pl.ds(H, H)], sems_ref.at[sl, 1]).start()

    @pl.when(i == 0)
    def _():
        issue(0, 0)

    @pl.when(i + 1 < 3 * NB)
    def _():
        issue(i + 1, 1 - slot)

    pltpu.make_async_copy(
        adj1_hbm.at[pl.ds(0, H), :],
        bufs_ref.at[slot, pl.ds(0, H)], sems_ref.at[slot, 0]).wait()
    pltpu.make_async_copy(
        adj1_hbm.at[pl.ds(0, H), :],
        bufs_ref.at[slot, pl.ds(H, H)], sems_ref.at[slot, 1]).wait()

    def mid(x_ref, w1_ref, b1_ref, w2_ref, s2_ref, k):
        a = bufs_ref[slot].astype(jnp.bfloat16)
        t = jnp.dot(a, x_ref[...], preferred_element_type=jnp.float32)
        t = jnp.dot(t.astype(jnp.bfloat16), w1_ref[...],
                    preferred_element_type=jnp.float32)
        h = jnp.maximum(t + b1_ref[...], 0.0).astype(jnp.bfloat16)
        s2_ref[pl.ds(k * BLK, BLK), :] = jnp.dot(
            h, w2_ref[...], preferred_element_type=jnp.float32
        ).astype(jnp.bfloat16)
        adjv_ref[pl.ds(k * BLK, BLK), :] = a

    def out(s2_ref, b2_ref, z_ref, k):
        z = jnp.dot(adjv_ref[pl.ds(k * BLK, BLK), :], s2_ref[...],
                    preferred_element_type=jnp.float32) + b2_ref[...]
        z_ref[...] = z
        nrm = jnp.sqrt(jnp.sum(z * z, axis=1, keepdims=True))
        return (z / nrm).astype(jnp.bfloat16)

    @pl.when(i < NB)
    def _():
        mid(x2_ref, w21_ref, b21_ref, w22_ref, s22_ref, i)

    @pl.when((NB <= i) & (i < 2 * NB))
    def _():
        k = i - NB
        zn2_ref[pl.ds(k * BLK, BLK), :] = out(s22_ref, b22_ref, z2_ref, k)
        mid(x1_ref, w11_ref, b11_ref, w12_ref, s21_ref, k)

    @pl.when(2 * NB <= i)
    def _():
        k = i - 2 * NB
        zn1 = out(s21_ref, b12_ref, z1_ref, k)
        cos = lax.dot_general(
            zn1, zn2_ref[...],
            dimension_numbers=(((1,), (1,)), ((), ())),
            preferred_element_type=jnp.float32)
        sim = jnp.exp(cos * (1.0 / TAU))
        s = jnp.sum(sim, axis=1, keepdims=True)
        w = jnp.sum(sim * bufs_ref[slot], axis=1, keepdims=True)
        part = jnp.sum(jnp.log(s + 1e-8) - jnp.log(w))

        @pl.when(i == 2 * NB)
        def _():
            acc_ref[...] = jnp.zeros_like(acc_ref)

        acc_ref[...] += part


def kernel(x1, adj1, x2, adj2, clm, W11, b11, W12, b12, W21, b21, W22, b22):
    bf = jnp.bfloat16
    z1, z2, acc = pl.pallas_call(
        _mega_body,
        grid=(3 * NB,),
        in_specs=[
            pl.BlockSpec(memory_space=pl.ANY),
            pl.BlockSpec(memory_space=pl.ANY),
            pl.BlockSpec(memory_space=pl.ANY),
            pl.BlockSpec((N, NFEAT), lambda i: (0, 0)),
            pl.BlockSpec((N, NFEAT), lambda i: (0, 0)),
            pl.BlockSpec((NFEAT, NFEAT), lambda i: (0, 0)),
            pl.BlockSpec((NFEAT, NFEAT), lambda i: (0, 0)),
            pl.BlockSpec((NFEAT, NHID), lambda i: (0, 0)),
            pl.BlockSpec((NFEAT, NHID), lambda i: (0, 0)),
            pl.BlockSpec((1, NFEAT), lambda i: (0, 0)),
            pl.BlockSpec((1, NFEAT), lambda i: (0, 0)),
            pl.BlockSpec((1, NHID), lambda i: (0, 0)),
            pl.BlockSpec((1, NHID), lambda i: (0, 0)),
        ],
        out_specs=(
            pl.BlockSpec((BLK, NHID), lambda i: (jnp.clip(i - 2 * NB, 0, NB - 1), 0)),
            pl.BlockSpec((BLK, NHID), lambda i: (jnp.clip(i - NB, 0, NB - 1), 0)),
            pl.BlockSpec((1, 1), lambda i: (0, 0)),
        ),
        out_shape=(
            jax.ShapeDtypeStruct((N, NHID), jnp.float32),
            jax.ShapeDtypeStruct((N, NHID), jnp.float32),
            jax.ShapeDtypeStruct((1, 1), jnp.float32),
        ),
        scratch_shapes=[
            pltpu.VMEM((2, BLK, N), jnp.float32),
            pltpu.VMEM((N, N), jnp.bfloat16),
            pltpu.VMEM((N, NHID), jnp.bfloat16),
            pltpu.VMEM((N, NHID), jnp.bfloat16),
            pltpu.VMEM((N, NHID), jnp.bfloat16),
            pltpu.SemaphoreType.DMA((2, 2)),
        ],
    )(adj1, adj2, clm,
      x1.astype(bf), x2.astype(bf), W11.astype(bf), W21.astype(bf),
      W12.astype(bf), W22.astype(bf),
      b11.reshape(1, -1), b21.reshape(1, -1),
      b12.reshape(1, -1), b22.reshape(1, -1))
    cl_loss = (acc[0, 0] / N).astype(jnp.float32).reshape(())
    return (z1, z2, cl_loss)


# speedup vs baseline: 1.0605x; 1.0605x over previous
import jax
import jax.numpy as jnp
from jax import lax
from jax.experimental import pallas as pl
from jax.experimental.pallas import tpu as pltpu

N = 4096
NFEAT = 256
NHID = 128
TAU = 0.5
BLK = 256
NB = N // BLK
NBUF = 3


def _mega_body(adj1_hbm, adj2_hbm, clm_hbm,
               x1_ref, x2_ref, w11_ref, w21_ref, w12_ref, w22_ref,
               b11_ref, b21_ref, b12_ref, b22_ref,
               z1_ref, z2_ref, acc_ref,
               bufs_ref, adjv_ref, s21_ref, s22_ref, zn2_ref, sems_ref):
    i = pl.program_id(0)
    slot = lax.rem(i, NBUF)

    def issue(j, sl):
        @pl.when(j < NB)
        def _():
            pltpu.make_async_copy(
                adj2_hbm.at[pl.ds(j * BLK, BLK), :],
                bufs_ref.at[sl], sems_ref.at[sl]).start()

        @pl.when((NB <= j) & (j < 2 * NB))
        def _():
            pltpu.make_async_copy(
                adj1_hbm.at[pl.ds((j - NB) * BLK, BLK), :],
                bufs_ref.at[sl], sems_ref.at[sl]).start()

        @pl.when(2 * NB <= j)
        def _():
            pltpu.make_async_copy(
                clm_hbm.at[pl.ds((j - 2 * NB) * BLK, BLK), :],
                bufs_ref.at[sl], sems_ref.at[sl]).start()

    @pl.when(i == 0)
    def _():
        issue(0, 0)
        issue(1, 1)

    @pl.when(i + 2 < 3 * NB)
    def _():
        issue(i + 2, lax.rem(i + 2, NBUF))

    pltpu.make_async_copy(
        adj1_hbm.at[pl.ds(0, BLK), :],
        bufs_ref.at[slot], sems_ref.at[slot]).wait()

    def mid(x_ref, w1_ref, b1_ref, w2_ref, s2_ref, k):
        a = bufs_ref[slot].astype(jnp.bfloat16)
        t = jnp.dot(a, x_ref[...], preferred_element_type=jnp.float32)
        t = jnp.dot(t.astype(jnp.bfloat16), w1_ref[...],
                    preferred_element_type=jnp.float32)
        h = jnp.maximum(t + b1_ref[...], 0.0).astype(jnp.bfloat16)
        s2_ref[pl.ds(k * BLK, BLK), :] = jnp.dot(
            h, w2_ref[...], preferred_element_type=jnp.float32
        ).astype(jnp.bfloat16)
        adjv_ref[pl.ds(k * BLK, BLK), :] = a

    def out(s2_ref, b2_ref, z_ref, k):
        z = jnp.dot(adjv_ref[pl.ds(k * BLK, BLK), :], s2_ref[...],
                    preferred_element_type=jnp.float32) + b2_ref[...]
        z_ref[...] = z
        nrm = jnp.sqrt(jnp.sum(z * z, axis=1, keepdims=True))
        return (z / nrm).astype(jnp.bfloat16)

    @pl.when(i < NB)
    def _():
        mid(x2_ref, w21_ref, b21_ref, w22_ref, s22_ref, i)

    @pl.when((NB <= i) & (i < 2 * NB))
    def _():
        k = i - NB
        zn2_ref[pl.ds(k * BLK, BLK), :] = out(s22_ref, b22_ref, z2_ref, k)
        mid(x1_ref, w11_ref, b11_ref, w12_ref, s21_ref, k)

    @pl.when(2 * NB <= i)
    def _():
        k = i - 2 * NB
        zn1 = out(s21_ref, b12_ref, z1_ref, k)
        cos = lax.dot_general(
            zn1, zn2_ref[...],
            dimension_numbers=(((1,), (1,)), ((), ())),
            preferred_element_type=jnp.float32)
        sim = jnp.exp(cos * (1.0 / TAU))
        s = jnp.sum(sim, axis=1, keepdims=True)
        w = jnp.sum(sim * bufs_ref[slot], axis=1, keepdims=True)
        part = jnp.sum(jnp.log(s + 1e-8) - jnp.log(w))

        @pl.when(i == 2 * NB)
        def _():
            acc_ref[...] = jnp.zeros_like(acc_ref)

        acc_ref[...] += part


def kernel(x1, adj1, x2, adj2, clm, W11, b11, W12, b12, W21, b21, W22, b22):
    bf = jnp.bfloat16
    z1, z2, acc = pl.pallas_call(
        _mega_body,
        grid=(3 * NB,),
        in_specs=[
            pl.BlockSpec(memory_space=pl.ANY),
            pl.BlockSpec(memory_space=pl.ANY),
            pl.BlockSpec(memory_space=pl.ANY),
            pl.BlockSpec((N, NFEAT), lambda i: (0, 0)),
            pl.BlockSpec((N, NFEAT), lambda i: (0, 0)),
            pl.BlockSpec((NFEAT, NFEAT), lambda i: (0, 0)),
            pl.BlockSpec((NFEAT, NFEAT), lambda i: (0, 0)),
            pl.BlockSpec((NFEAT, NHID), lambda i: (0, 0)),
            pl.BlockSpec((NFEAT, NHID), lambda i: (0, 0)),
            pl.BlockSpec((1, NFEAT), lambda i: (0, 0)),
            pl.BlockSpec((1, NFEAT), lambda i: (0, 0)),
            pl.BlockSpec((1, NHID), lambda i: (0, 0)),
            pl.BlockSpec((1, NHID), lambda i: (0, 0)),
        ],
        out_specs=(
            pl.BlockSpec((BLK, NHID), lambda i: (jnp.clip(i - 2 * NB, 0, NB - 1), 0)),
            pl.BlockSpec((BLK, NHID), lambda i: (jnp.clip(i - NB, 0, NB - 1), 0)),
            pl.BlockSpec((1, 1), lambda i: (0, 0)),
        ),
        out_shape=(
            jax.ShapeDtypeStruct((N, NHID), jnp.float32),
            jax.ShapeDtypeStruct((N, NHID), jnp.float32),
            jax.ShapeDtypeStruct((1, 1), jnp.float32),
        ),
        scratch_shapes=[
            pltpu.VMEM((NBUF, BLK, N), jnp.float32),
            pltpu.VMEM((N, N), jnp.bfloat16),
            pltpu.VMEM((N, NHID), jnp.bfloat16),
            pltpu.VMEM((N, NHID), jnp.bfloat16),
            pltpu.VMEM((N, NHID), jnp.bfloat16),
            pltpu.SemaphoreType.DMA((NBUF,)),
        ],
    )(adj1, adj2, clm,
      x1.astype(bf), x2.astype(bf), W11.astype(bf), W21.astype(bf),
      W12.astype(bf), W22.astype(bf),
      b11.reshape(1, -1), b21.reshape(1, -1),
      b12.reshape(1, -1), b22.reshape(1, -1))
    cl_loss = (acc[0, 0] / N).astype(jnp.float32).reshape(())
    return (z1, z2, cl_loss)
